# Initial kernel scaffold; baseline (speedup 1.0000x reference)
import jax
import jax.numpy as jnp
from jax.experimental import pallas as pl


def kernel(xyz, feature, relative_knn_xyz, knn_idx, W_d1, b_d1, W_d2, b_d2, W_fc1, b_fc1, W_q, W_k, W_v, W_g1, b_g1, W_g2, b_g2, W_fc2, b_fc2, W_sc, b_sc):
    raise NotImplementedError("write your pallas kernel here")


# speedup vs baseline: 12.4310x; 12.4310x over previous
import functools
import math

import jax
import jax.numpy as jnp
from jax import lax
from jax.experimental import pallas as pl
from jax.experimental.pallas import tpu as pltpu
from jax.experimental.pallas import tpu_sc as plsc


def _table_body(feat_ref, w_ref, b_ref, out_ref):
    out_ref[:] = (
        jnp.dot(feat_ref[:], w_ref[:], preferred_element_type=jnp.float32)
        + b_ref[:]
    )


def _compute_table(feature2, w_fc1, b_fc1_row):
    n, d_in = feature2.shape
    d_model = w_fc1.shape[1]
    return pl.pallas_call(
        _table_body,
        out_shape=jax.ShapeDtypeStruct((n, d_model), jnp.float32),
    )(feature2, w_fc1, b_fc1_row)


def _sc_gather(idx_flat, table):
    nk = idx_flat.shape[0]
    d = table.shape[1]
    info = plsc.get_sparse_core_info()
    nw = info.num_cores * info.num_subcores
    per_w = nk // nw
    assert per_w * nw == nk and per_w % 8 == 0
    chunk = 400
    assert per_w % chunk == 0
    n_chunks = per_w // chunk
    mesh = plsc.VectorSubcoreMesh(core_axis_name="c", subcore_axis_name="s")

    @functools.partial(
        pl.kernel,
        out_type=jax.ShapeDtypeStruct((nk, d), jnp.float32),
        mesh=mesh,
        scratch_types=[
            pltpu.VMEM((chunk,), jnp.int32),
            pltpu.VMEM((chunk, d), jnp.float32),
            pltpu.SemaphoreType.DMA,
        ],
    )
    def gather_kernel(idx_hbm, table_hbm, out_hbm, idx_v, rows_v, sem):
        wid = lax.axis_index("s") * info.num_cores + lax.axis_index("c")
        base = wid * per_w

        def body(i, carry):
            off = base + i * chunk
            pltpu.sync_copy(idx_hbm.at[pl.ds(off, chunk)], idx_v)
            pltpu.async_copy(table_hbm.at[idx_v], rows_v, sem).wait()
            pltpu.sync_copy(rows_v, out_hbm.at[pl.ds(off, chunk)])
            return carry

        lax.fori_loop(0, n_chunks, body, 0)

    return gather_kernel(idx_flat, table)


def _attn_body(relx_ref, rely_ref, relz_ref, knnf_ref, f_ref, feat_ref,
               wd1x_ref, wd1y_ref, wd1z_ref, bd1_ref, wd2_ref, bd2_ref,
               wq_ref, wk_ref, wv_ref, wg1_ref, bg1_ref, wg2_ref, bg2_ref,
               wfc2_ref, bfc2_ref, wsc_ref, bsc_ref, out_ref, *, nb, kk, dm):
    nbk = nb * kk
    f32 = jnp.float32

    a3 = (
        relx_ref[:][:, :, None] * wd1x_ref[:][None, :, :]
        + rely_ref[:][:, :, None] * wd1y_ref[:][None, :, :]
        + relz_ref[:][:, :, None] * wd1z_ref[:][None, :, :]
        + bd1_ref[:][None, :, :]
    )
    a2 = jnp.maximum(a3, 0.0).reshape(nbk, dm)
    pos = jnp.dot(a2, wd2_ref[:], preferred_element_type=f32) + bd2_ref[:]

    kf = knnf_ref[:]
    q = jnp.dot(f_ref[:], wq_ref[:], preferred_element_type=f32)
    k_ = jnp.dot(kf, wk_ref[:], preferred_element_type=f32)
    v = jnp.dot(kf, wv_ref[:], preferred_element_type=f32)

    q_full = jnp.broadcast_to(q[:, None, :], (nb, kk, dm)).reshape(nbk, dm)
    pre = q_full - k_ + pos
    h = jnp.dot(
        jnp.maximum(jnp.dot(pre, wg1_ref[:], preferred_element_type=f32)
                    + bg1_ref[:], 0.0),
        wg2_ref[:], preferred_element_type=f32,
    ) + bg2_ref[:]
    h = h * (1.0 / math.sqrt(dm))

    h3 = h.reshape(nb, kk, dm)
    m = jnp.max(h3, axis=1, keepdims=True)
    e = jnp.exp(h3 - m)
    s = jnp.sum(e, axis=1, keepdims=True)
    attn3 = e / s

    w3 = attn3 * (v + pos).reshape(nb, kk, dm)
    feat_out = jnp.sum(w3, axis=1)

    out_ref[:] = (
        jnp.dot(feat_out, wfc2_ref[:], preferred_element_type=f32)
        + bfc2_ref[:]
        + jnp.dot(feat_ref[:], wsc_ref[:], preferred_element_type=f32)
        + bsc_ref[:]
    )


def _attn_call(relx, rely, relz, knnf, f, feature2, wd1x, wd1y, wd1z, bd1,
               wd2, bd2, wq, wk, wv, wg1, bg1, wg2, bg2, wfc2, bfc2, wsc, bsc,
               nb):
    n, kk = relx.shape
    dm = wd2.shape[0]
    d_out = wfc2.shape[1]
    grid = n // nb
    assert grid * nb == n

    def blk(i):
        return (i, 0)

    def full(i):
        return (0, 0)

    nk_spec = pl.BlockSpec((nb, kk), blk)
    row_spec = pl.BlockSpec((nb, dm), blk)
    edge_spec = pl.BlockSpec((nb * kk, dm), blk)

    def w_spec(a):
        return pl.BlockSpec(a.shape, full if a.ndim == 2 else None)

    body = functools.partial(_attn_body, nb=nb, kk=kk, dm=dm)
    return pl.pallas_call(
        body,
        grid=(grid,),
        in_specs=[
            nk_spec, nk_spec, nk_spec, edge_spec,
            pl.BlockSpec((nb, dm), blk), pl.BlockSpec((nb, feature2.shape[1]), blk),
            w_spec(wd1x), w_spec(wd1y), w_spec(wd1z), w_spec(bd1),
            w_spec(wd2), w_spec(bd2), w_spec(wq), w_spec(wk), w_spec(wv),
            w_spec(wg1), w_spec(bg1), w_spec(wg2), w_spec(bg2),
            w_spec(wfc2), w_spec(bfc2), w_spec(wsc), w_spec(bsc),
        ],
        out_specs=pl.BlockSpec((nb, d_out), blk),
        out_shape=jax.ShapeDtypeStruct((n, d_out), jnp.float32),
    )(relx, rely, relz, knnf, f, feature2, wd1x, wd1y, wd1z, bd1, wd2, bd2,
      wq, wk, wv, wg1, bg1, wg2, bg2, wfc2, bfc2, wsc, bsc)


def kernel(xyz, feature, relative_knn_xyz, knn_idx, W_d1, b_d1, W_d2, b_d2,
           W_fc1, b_fc1, W_q, W_k, W_v, W_g1, b_g1, W_g2, b_g2,
           W_fc2, b_fc2, W_sc, b_sc):
    feature2 = feature[0]
    relx = relative_knn_xyz[0, :, :, 0]
    rely = relative_knn_xyz[0, :, :, 1]
    relz = relative_knn_xyz[0, :, :, 2]
    idx_flat = knn_idx[0].reshape(-1)

    f = _compute_table(feature2, W_fc1, b_fc1[None, :])
    knnf = _sc_gather(idx_flat, f)

    feat = _attn_call(
        relx, rely, relz, knnf, f, feature2,
        W_d1[0:1, :], W_d1[1:2, :], W_d1[2:3, :], b_d1[None, :],
        W_d2, b_d2[None, :], W_q, W_k, W_v,
        W_g1, b_g1[None, :], W_g2, b_g2[None, :],
        W_fc2, b_fc2[None, :], W_sc, b_sc[None, :],
        nb=200,
    )
    return (xyz, feat[None], relative_knn_xyz, knn_idx)


# speedup vs baseline: 12.5879x; 1.0126x over previous
import functools
import math

import jax
import jax.numpy as jnp
from jax import lax
from jax.experimental import pallas as pl
from jax.experimental.pallas import tpu as pltpu
from jax.experimental.pallas import tpu_sc as plsc


def _table_body(feat_ref, w_ref, b_ref, out_ref):
    out_ref[:] = (
        jnp.dot(feat_ref[:], w_ref[:], preferred_element_type=jnp.float32)
        + b_ref[:]
    )


def _compute_table(feature2, w_fc1, b_fc1_row):
    n, d_in = feature2.shape
    d_model = w_fc1.shape[1]
    return pl.pallas_call(
        _table_body,
        out_shape=jax.ShapeDtypeStruct((n, d_model), jnp.float32),
    )(feature2, w_fc1, b_fc1_row)


def _sc_gather(idx_flat, table):
    nk = idx_flat.shape[0]
    d = table.shape[1]
    info = plsc.get_sparse_core_info()
    nw = info.num_cores * info.num_subcores
    per_w = nk // nw
    assert per_w * nw == nk and per_w % 8 == 0
    chunk = 400
    assert per_w % chunk == 0
    n_chunks = per_w // chunk
    mesh = plsc.VectorSubcoreMesh(core_axis_name="c", subcore_axis_name="s")

    @functools.partial(
        pl.kernel,
        out_type=jax.ShapeDtypeStruct((nk, d), jnp.float32),
        mesh=mesh,
        scratch_types=[
            pltpu.VMEM((chunk,), jnp.int32),
            pltpu.VMEM((chunk, d), jnp.float32),
            pltpu.SemaphoreType.DMA,
        ],
    )
    def gather_kernel(idx_hbm, table_hbm, out_hbm, idx_v, rows_v, sem):
        wid = lax.axis_index("s") * info.num_cores + lax.axis_index("c")
        base = wid * per_w

        def body(i, carry):
            off = base + i * chunk
            pltpu.sync_copy(idx_hbm.at[pl.ds(off, chunk)], idx_v)
            pltpu.async_copy(table_hbm.at[idx_v], rows_v, sem).wait()
            pltpu.sync_copy(rows_v, out_hbm.at[pl.ds(off, chunk)])
            return carry

        lax.fori_loop(0, n_chunks, body, 0)

    return gather_kernel(idx_flat, table)


def _attn_body(relx_ref, rely_ref, relz_ref, knnf_ref, f_ref, feat_ref,
               wd1x_ref, wd1y_ref, wd1z_ref, bd1_ref, wd2_ref, bd2_ref,
               wq_ref, wk_ref, wv_ref, wg1_ref, bg1_ref, wg2_ref, bg2_ref,
               wfc2_ref, bfc2_ref, wsc_ref, bsc_ref, out_ref, *, nb, kk, dm):
    nbk = nb * kk
    f32 = jnp.float32
    bf16 = jnp.bfloat16

    a3 = (
        relx_ref[:][:, :, None] * wd1x_ref[:][None, :, :]
        + rely_ref[:][:, :, None] * wd1y_ref[:][None, :, :]
        + relz_ref[:][:, :, None] * wd1z_ref[:][None, :, :]
        + bd1_ref[:][None, :, :]
    )
    a2 = jnp.maximum(a3, 0.0).reshape(nbk, dm).astype(bf16)
    pos = jnp.dot(a2, wd2_ref[:].astype(bf16),
                  preferred_element_type=f32) + bd2_ref[:]

    kf = knnf_ref[:].astype(bf16)
    q = jnp.dot(f_ref[:].astype(bf16), wq_ref[:].astype(bf16),
                preferred_element_type=f32)
    wkv = jnp.concatenate(
        [wk_ref[:].astype(bf16), wv_ref[:].astype(bf16)], axis=1)
    kv = jnp.dot(kf, wkv, preferred_element_type=f32)
    k_ = kv[:, :dm]
    v = kv[:, dm:]

    q_full = jnp.broadcast_to(q[:, None, :], (nb, kk, dm)).reshape(nbk, dm)
    pre = (q_full - k_ + pos).astype(bf16)
    h = jnp.dot(
        jnp.maximum(jnp.dot(pre, wg1_ref[:].astype(bf16),
                            preferred_element_type=f32)
                    + bg1_ref[:], 0.0).astype(bf16),
        wg2_ref[:].astype(bf16), preferred_element_type=f32,
    ) + bg2_ref[:]
    h = h * (1.0 / math.sqrt(dm))

    h3 = h.reshape(nb, kk, dm)
    m = jnp.max(h3, axis=1, keepdims=True)
    e = jnp.exp(h3 - m)
    s = jnp.sum(e, axis=1, keepdims=True)
    attn3 = e / s

    w3 = attn3 * (v + pos).reshape(nb, kk, dm)
    feat_out = jnp.sum(w3, axis=1)

    out_ref[:] = (
        jnp.dot(feat_out.astype(bf16), wfc2_ref[:].astype(bf16),
                preferred_element_type=f32)
        + bfc2_ref[:]
        + jnp.dot(feat_ref[:].astype(bf16), wsc_ref[:].astype(bf16),
                  preferred_element_type=f32)
        + bsc_ref[:]
    )


def _attn_call(relx, rely, relz, knnf, f, feature2, wd1x, wd1y, wd1z, bd1,
               wd2, bd2, wq, wk, wv, wg1, bg1, wg2, bg2, wfc2, bfc2, wsc, bsc,
               nb):
    n, kk = relx.shape
    dm = wd2.shape[0]
    d_out = wfc2.shape[1]
    grid = n // nb
    assert grid * nb == n

    def blk(i):
        return (i, 0)

    def full(i):
        return (0, 0)

    nk_spec = pl.BlockSpec((nb, kk), blk)
    row_spec = pl.BlockSpec((nb, dm), blk)
    edge_spec = pl.BlockSpec((nb * kk, dm), blk)

    def w_spec(a):
        return pl.BlockSpec(a.shape, full if a.ndim == 2 else None)

    body = functools.partial(_attn_body, nb=nb, kk=kk, dm=dm)
    return pl.pallas_call(
        body,
        grid=(grid,),
        in_specs=[
            nk_spec, nk_spec, nk_spec, edge_spec,
            pl.BlockSpec((nb, dm), blk), pl.BlockSpec((nb, feature2.shape[1]), blk),
            w_spec(wd1x), w_spec(wd1y), w_spec(wd1z), w_spec(bd1),
            w_spec(wd2), w_spec(bd2), w_spec(wq), w_spec(wk), w_spec(wv),
            w_spec(wg1), w_spec(bg1), w_spec(wg2), w_spec(bg2),
            w_spec(wfc2), w_spec(bfc2), w_spec(wsc), w_spec(bsc),
        ],
        out_specs=pl.BlockSpec((nb, d_out), blk),
        out_shape=jax.ShapeDtypeStruct((n, d_out), jnp.float32),
    )(relx, rely, relz, knnf, f, feature2, wd1x, wd1y, wd1z, bd1, wd2, bd2,
      wq, wk, wv, wg1, bg1, wg2, bg2, wfc2, bfc2, wsc, bsc)


def kernel(xyz, feature, relative_knn_xyz, knn_idx, W_d1, b_d1, W_d2, b_d2,
           W_fc1, b_fc1, W_q, W_k, W_v, W_g1, b_g1, W_g2, b_g2,
           W_fc2, b_fc2, W_sc, b_sc):
    feature2 = feature[0]
    relx = relative_knn_xyz[0, :, :, 0]
    rely = relative_knn_xyz[0, :, :, 1]
    relz = relative_knn_xyz[0, :, :, 2]
    idx_flat = knn_idx[0].reshape(-1)

    f = _compute_table(feature2, W_fc1, b_fc1[None, :])
    knnf = _sc_gather(idx_flat, f)

    feat = _attn_call(
        relx, rely, relz, knnf, f, feature2,
        W_d1[0:1, :], W_d1[1:2, :], W_d1[2:3, :], b_d1[None, :],
        W_d2, b_d2[None, :], W_q, W_k, W_v,
        W_g1, b_g1[None, :], W_g2, b_g2[None, :],
        W_fc2, b_fc2[None, :], W_sc, b_sc[None, :],
        nb=200,
    )
    return (xyz, feat[None], relative_knn_xyz, knn_idx)
